# speedup vs baseline: 1.5397x; 1.5397x over previous
import functools

import jax
import jax.numpy as jnp
from jax import lax
from jax.experimental import pallas as pl
from jax.experimental.pallas import tpu as pltpu
from jax.experimental.pallas import tpu_sc as plsc

D_VOCAB = 100000
D_MODEL = 1024
BATCH = 4
SEQ = 4096
N_TOKENS = BATCH * SEQ

_info = plsc.get_sparse_core_info()
NC = _info.num_cores
NS = _info.num_subcores
NW = NC * NS
B_PER_W = N_TOKENS // NW
CHUNK = 64
N_CHUNKS = B_PER_W // CHUNK


def _emb_body(tok_hbm, table_hbm, out_hbm, idx_v, buf_v, sem):
    wid = lax.axis_index("s") * NC + lax.axis_index("c")
    base = wid * B_PER_W
    pltpu.sync_copy(tok_hbm.at[pl.ds(base, B_PER_W)], idx_v)
    for c in range(N_CHUNKS):
        off = c * CHUNK
        pltpu.async_copy(
            table_hbm.at[idx_v.at[pl.ds(off, CHUNK)]], buf_v, sem
        ).wait()
        pltpu.sync_copy(buf_v, out_hbm.at[pl.ds(base + off, CHUNK)])


@functools.partial(
    pl.kernel,
    mesh=plsc.VectorSubcoreMesh(core_axis_name="c", subcore_axis_name="s"),
    out_type=jax.ShapeDtypeStruct((N_TOKENS, D_MODEL), jnp.float32),
    scratch_types=[
        pltpu.VMEM((B_PER_W,), jnp.int32),
        pltpu.VMEM((CHUNK, D_MODEL), jnp.float32),
        pltpu.SemaphoreType.DMA,
    ],
)
def _emb_lookup(tok_hbm, table_hbm, out_hbm, idx_v, buf_v, sem):
    _emb_body(tok_hbm, table_hbm, out_hbm, idx_v, buf_v, sem)


def kernel(tokens, table):
    flat = jnp.reshape(tokens, (N_TOKENS,)).astype(jnp.int32)
    out = _emb_lookup(flat, table)
    return jnp.reshape(out, (BATCH, SEQ, D_MODEL))


# speedup vs baseline: 1.6331x; 1.0607x over previous
import functools

import jax
import jax.numpy as jnp
from jax import lax
from jax.experimental import pallas as pl
from jax.experimental.pallas import tpu as pltpu
from jax.experimental.pallas import tpu_sc as plsc

D_VOCAB = 100000
D_MODEL = 1024
BATCH = 4
SEQ = 4096
N_TOKENS = BATCH * SEQ

_info = plsc.get_sparse_core_info()
NC = _info.num_cores
NS = _info.num_subcores
NW = NC * NS
B_PER_W = N_TOKENS // NW
CHUNK = 32
N_CHUNKS = B_PER_W // CHUNK


def _emb_body(tok_hbm, table_hbm, out_hbm, idx_v,
              buf0, buf1, gsem0, gsem1, wsem0, wsem1):
    wid = lax.axis_index("s") * NC + lax.axis_index("c")
    base = wid * B_PER_W
    pltpu.sync_copy(tok_hbm.at[pl.ds(base, B_PER_W)], idx_v)
    bufs = (buf0, buf1)
    gsems = (gsem0, gsem1)
    wsems = (wsem0, wsem1)

    def start_gather(c, b):
        return pltpu.async_copy(
            table_hbm.at[idx_v.at[pl.ds(c * CHUNK, CHUNK)]], bufs[b], gsems[b])

    def start_writeback(c, b):
        return pltpu.async_copy(
            bufs[b], out_hbm.at[pl.ds(base + c * CHUNK, CHUNK)], wsems[b])

    gh = [start_gather(0, 0), None]
    wh = [None, None]
    for c in range(N_CHUNKS):
        b = c & 1
        nb = 1 - b
        if c + 1 < N_CHUNKS:
            if wh[nb] is not None:
                wh[nb].wait()
            gh[nb] = start_gather(c + 1, nb)
        gh[b].wait()
        wh[b] = start_writeback(c, b)
    wh[0].wait()
    wh[1].wait()


@functools.partial(
    pl.kernel,
    mesh=plsc.VectorSubcoreMesh(core_axis_name="c", subcore_axis_name="s"),
    out_type=jax.ShapeDtypeStruct((N_TOKENS, D_MODEL), jnp.float32),
    scratch_types=[
        pltpu.VMEM((B_PER_W,), jnp.int32),
        pltpu.VMEM((CHUNK, D_MODEL), jnp.float32),
        pltpu.VMEM((CHUNK, D_MODEL), jnp.float32),
        pltpu.SemaphoreType.DMA,
        pltpu.SemaphoreType.DMA,
        pltpu.SemaphoreType.DMA,
        pltpu.SemaphoreType.DMA,
    ],
)
def _emb_lookup(tok_hbm, table_hbm, out_hbm, idx_v,
                buf0, buf1, gsem0, gsem1, wsem0, wsem1):
    _emb_body(tok_hbm, table_hbm, out_hbm, idx_v,
              buf0, buf1, gsem0, gsem1, wsem0, wsem1)


def kernel(tokens, table):
    flat = jnp.reshape(tokens, (N_TOKENS,)).astype(jnp.int32)
    out = _emb_lookup(flat, table)
    return jnp.reshape(out, (BATCH, SEQ, D_MODEL))


# speedup vs baseline: 1.6616x; 1.0175x over previous
import functools

import jax
import jax.numpy as jnp
from jax import lax
from jax.experimental import pallas as pl
from jax.experimental.pallas import tpu as pltpu
from jax.experimental.pallas import tpu_sc as plsc

D_VOCAB = 100000
D_MODEL = 1024
BATCH = 4
SEQ = 4096
N_TOKENS = BATCH * SEQ

_info = plsc.get_sparse_core_info()
NC = _info.num_cores
NS = _info.num_subcores
NW = NC * NS
B_PER_W = N_TOKENS // NW
CHUNK = 32
N_CHUNKS = B_PER_W // CHUNK


NBUF = 3


def _emb_body(tok_hbm, table_hbm, out_hbm, idx_v, bufs, gsems, wsems):
    wid = lax.axis_index("s") * NC + lax.axis_index("c")
    base = wid * B_PER_W
    pltpu.sync_copy(tok_hbm.at[pl.ds(base, B_PER_W)], idx_v)

    def start_gather(c, b):
        return pltpu.async_copy(
            table_hbm.at[idx_v.at[pl.ds(c * CHUNK, CHUNK)]], bufs[b], gsems[b])

    def start_writeback(c, b):
        return pltpu.async_copy(
            bufs[b], out_hbm.at[pl.ds(base + c * CHUNK, CHUNK)], wsems[b])

    gh = [None] * NBUF
    wh = [None] * NBUF
    for c in range(min(NBUF - 1, N_CHUNKS)):
        gh[c % NBUF] = start_gather(c, c % NBUF)
    for c in range(N_CHUNKS):
        b = c % NBUF
        nc = c + NBUF - 1
        if nc < N_CHUNKS:
            nb = nc % NBUF
            if wh[nb] is not None:
                wh[nb].wait()
            gh[nb] = start_gather(nc, nb)
        gh[b].wait()
        wh[b] = start_writeback(c, b)
    for b in range(NBUF):
        if wh[b] is not None:
            wh[b].wait()


@functools.partial(
    pl.kernel,
    mesh=plsc.VectorSubcoreMesh(core_axis_name="c", subcore_axis_name="s"),
    out_type=jax.ShapeDtypeStruct((N_TOKENS, D_MODEL), jnp.float32),
    scratch_types=(
        [pltpu.VMEM((B_PER_W,), jnp.int32)]
        + [pltpu.VMEM((CHUNK, D_MODEL), jnp.float32)] * NBUF
        + [pltpu.SemaphoreType.DMA] * (2 * NBUF)
    ),
)
def _emb_lookup(tok_hbm, table_hbm, out_hbm, idx_v, *rest):
    bufs = rest[:NBUF]
    gsems = rest[NBUF:2 * NBUF]
    wsems = rest[2 * NBUF:3 * NBUF]
    _emb_body(tok_hbm, table_hbm, out_hbm, idx_v, bufs, gsems, wsems)


def kernel(tokens, table):
    flat = jnp.reshape(tokens, (N_TOKENS,)).astype(jnp.int32)
    out = _emb_lookup(flat, table)
    return jnp.reshape(out, (BATCH, SEQ, D_MODEL))


# speedup vs baseline: 1.6681x; 1.0039x over previous
import functools

import jax
import jax.numpy as jnp
from jax import lax
from jax.experimental import pallas as pl
from jax.experimental.pallas import tpu as pltpu
from jax.experimental.pallas import tpu_sc as plsc

D_VOCAB = 100000
D_MODEL = 1024
BATCH = 4
SEQ = 4096
N_TOKENS = BATCH * SEQ

_info = plsc.get_sparse_core_info()
NC = _info.num_cores
NS = _info.num_subcores
NW = NC * NS
B_PER_W = N_TOKENS // NW
CHUNK = 16
N_CHUNKS = B_PER_W // CHUNK


NBUF = 6


def _emb_body(tok_hbm, table_hbm, out_hbm, idx_v, bufs, gsems, wsems):
    wid = lax.axis_index("s") * NC + lax.axis_index("c")
    base = wid * B_PER_W
    pltpu.sync_copy(tok_hbm.at[pl.ds(base, B_PER_W)], idx_v)

    def start_gather(c, b):
        return pltpu.async_copy(
            table_hbm.at[idx_v.at[pl.ds(c * CHUNK, CHUNK)]], bufs[b], gsems[b])

    def start_writeback(c, b):
        return pltpu.async_copy(
            bufs[b], out_hbm.at[pl.ds(base + c * CHUNK, CHUNK)], wsems[b])

    gh = [None] * NBUF
    wh = [None] * NBUF
    for c in range(min(NBUF - 1, N_CHUNKS)):
        gh[c % NBUF] = start_gather(c, c % NBUF)
    for c in range(N_CHUNKS):
        b = c % NBUF
        nc = c + NBUF - 1
        if nc < N_CHUNKS:
            nb = nc % NBUF
            if wh[nb] is not None:
                wh[nb].wait()
            gh[nb] = start_gather(nc, nb)
        gh[b].wait()
        wh[b] = start_writeback(c, b)
    for b in range(NBUF):
        if wh[b] is not None:
            wh[b].wait()


@functools.partial(
    pl.kernel,
    mesh=plsc.VectorSubcoreMesh(core_axis_name="c", subcore_axis_name="s"),
    out_type=jax.ShapeDtypeStruct((N_TOKENS, D_MODEL), jnp.float32),
    scratch_types=(
        [pltpu.VMEM((B_PER_W,), jnp.int32)]
        + [pltpu.VMEM((CHUNK, D_MODEL), jnp.float32)] * NBUF
        + [pltpu.SemaphoreType.DMA] * (2 * NBUF)
    ),
)
def _emb_lookup(tok_hbm, table_hbm, out_hbm, idx_v, *rest):
    bufs = rest[:NBUF]
    gsems = rest[NBUF:2 * NBUF]
    wsems = rest[2 * NBUF:3 * NBUF]
    _emb_body(tok_hbm, table_hbm, out_hbm, idx_v, bufs, gsems, wsems)


def kernel(tokens, table):
    flat = jnp.reshape(tokens, (N_TOKENS,)).astype(jnp.int32)
    out = _emb_lookup(flat, table)
    return jnp.reshape(out, (BATCH, SEQ, D_MODEL))
